# baseline (speedup 1.0000x reference)
import functools

import jax
import jax.numpy as jnp
from jax import lax
from jax.experimental import pallas as pl
from jax.experimental.pallas import tpu as pltpu
import jax.experimental.pallas.tpu_sc as plsc

N = 10000
E = 320000
DIM = 3
HID = 128
LAT = 128
OUT = 3

NP = 10112
NC, NS = 2, 16
NW = NC * NS
K = 128
RPT = NP // NS

FRAC0 = 0.9


def _ceil4(x):
    return max(4, -(-x // 4) * 4)


_S0 = int(round(FRAC0 * E))
CH0 = _ceil4(-(-_S0 // (NS * K)))
CH1 = _ceil4(-(-(E - _S0) // (NS * K)))
CHM = max(CH0, CH1)
ECH = NS * (CH0 + CH1)
ECHX = ECH + CHM

@functools.lru_cache(maxsize=None)
def _mesh():
    return plsc.VectorSubcoreMesh(core_axis_name="c", subcore_axis_name="s",
                                  num_cores=NC, num_subcores=NS)


def _selu(x):
    alpha = 1.6732632423543772848170429916717
    scale = 1.0507009873554804934193349852946
    return scale * jnp.where(x > 0, x, alpha * (jnp.exp(x) - 1.0))


@functools.lru_cache(maxsize=None)
def _deg_kernel_build():
    return pl.kernel(
        _deg_body,
        out_type=jax.ShapeDtypeStruct((NW, NP), jnp.float32),
        mesh=_mesh(),
        scratch_types=[
            pltpu.VMEM((NP,), jnp.float32),
            pltpu.VMEM((CHM, K), jnp.int32),
        ],
        compiler_params=pltpu.CompilerParams(needs_layout_passes=False),
    )


def _deg_kernel(dst2):
    return _deg_kernel_build()(dst2)


def _chunk_base(c, s):
    return jnp.where(c == 0, s * CH0, NS * CH0 + s * CH1)


def _deg_body(dst_hbm, out_hbm, hist_v, dstv):
    c = lax.axis_index("c")
    s = lax.axis_index("s")
    wid = c * NS + s
    cbase = _chunk_base(c, s)
    chn = jnp.where(c == 0, CH0, CH1)

    pltpu.sync_copy(dst_hbm.at[pl.ds(cbase, CHM)], dstv)

    def zb(k, _):
        hist_v[pl.ds(k * 16, 16)] = jnp.zeros((16,), jnp.float32)
        return 0

    lax.fori_loop(0, NP // 16, zb, 0)
    ones = jnp.ones((16,), jnp.float32)

    kg = K // 16

    def hb(k, _):
        i = lax.div(k, kg)
        j = lax.mul(lax.rem(k, kg), 16)
        idx = dstv[i, pl.ds(j, 16)]
        plsc.addupdate_scatter(hist_v, [idx], ones)
        return 0

    lax.fori_loop(0, chn * kg, hb, 0)
    pltpu.sync_copy(hist_v, out_hbm.at[wid])


@functools.lru_cache(maxsize=None)
def _make_agg(D):
    G = D // 16

    def agg_body(y_hbm, src_hbm, dst_hbm, out_hbm, acc_sh, sidx, didx,
                 rows0, rows1, isem0, isem1, isem2, isem3, gsem0, gsem1):
        c = lax.axis_index("c")
        s = lax.axis_index("s")
        cbase = _chunk_base(c, s)
        chn = jnp.where(c == 0, CH0, CH1)
        isem = (isem0, isem1, isem2, isem3)
        gsem = (gsem0, gsem1)
        rows = (rows0, rows1)

        def issue_idx(ch, b):
            pltpu.async_copy(
                src_hbm.at[pl.ds(lax.mul(cbase + ch, K), K)],
                sidx.at[b], isem[b])
            pltpu.async_copy(dst_hbm.at[cbase + ch], didx.at[b], isem[b])

        def wait_idx(b):
            pltpu.make_async_copy(
                src_hbm.at[pl.ds(0, K)], sidx.at[b], isem[b]).wait()
            pltpu.make_async_copy(
                dst_hbm.at[0], didx.at[b], isem[b]).wait()

        def issue_gather(b4, rb):
            pltpu.async_copy(y_hbm.at[sidx.at[b4]], rows[rb], gsem[rb])

        def wait_gather(b4, rb):
            pltpu.make_async_copy(
                y_hbm.at[sidx.at[b4]], rows[rb], gsem[rb]).wait()

        for b in range(4):
            issue_idx(b, b)

        def zb(k, _):
            i = lax.div(k, G)
            j = lax.mul(lax.rem(k, G), 16)
            rows0[i, pl.ds(j, 16)] = jnp.zeros((16,), jnp.float32)
            return 0

        lax.fori_loop(0, K * G, zb, 0)
        r0 = s * RPT
        for t in range(RPT // K):
            pltpu.sync_copy(rows0, acc_sh.at[pl.ds(r0 + t * K, K)])
        rem = RPT % K
        if rem:
            pltpu.sync_copy(rows0.at[pl.ds(0, rem)],
                            acc_sh.at[pl.ds(r0 + (RPT // K) * K, rem)])
        plsc.subcore_barrier()

        wait_idx(0)
        issue_gather(0, 0)
        wait_idx(1)
        issue_gather(1, 1)

        def body(p, _):
            for b in range(4):
                ch = 4 * p + b
                rb = b & 1
                wait_gather(b, rb)
                pltpu.sync_copy(rows[rb], acc_sh.at[didx.at[b]],
                                add=True)

                @pl.when(ch + 2 < chn)
                def _():
                    b2 = (b + 2) & 3
                    wait_idx(b2)
                    issue_gather(b2, rb)

                @pl.when(ch + 4 < chn)
                def _():
                    issue_idx(ch + 4, b)
            return 0

        lax.fori_loop(0, lax.div(chn, 4), body, 0)
        plsc.subcore_barrier()
        off = 0
        for sz in (K, K, K, K, RPT - 4 * K):
            pltpu.sync_copy(acc_sh.at[pl.ds(r0 + off, sz)],
                            rows0.at[pl.ds(0, sz)])
            pltpu.sync_copy(rows0.at[pl.ds(0, sz)],
                            out_hbm.at[c].at[pl.ds(r0 + off, sz)])
            off += sz

    return pl.kernel(
        agg_body,
        out_type=jax.ShapeDtypeStruct((NC, NP, D), jnp.float32),
        mesh=_mesh(),
        scratch_types=[
            pltpu.VMEM_SHARED((NP, D), jnp.float32),
            pltpu.VMEM((4, K), jnp.int32),
            pltpu.VMEM((4, K), jnp.int32),
            pltpu.VMEM((K, D), jnp.float32),
            pltpu.VMEM((K, D), jnp.float32),
            pltpu.SemaphoreType.DMA,
            pltpu.SemaphoreType.DMA,
            pltpu.SemaphoreType.DMA,
            pltpu.SemaphoreType.DMA,
            pltpu.SemaphoreType.DMA,
            pltpu.SemaphoreType.DMA,
        ],
    )


def _agg128(y, src, dst):
    return _make_agg(128)(y, src, dst)


def _agg16(y, src, dst):
    return _make_agg(16)(y, src, dst)


_R = 1264
_GRID = NP // _R


def _a0_body(lat_ref, pos_ref, deg_ref, wlin_ref, blin_ref, wf0_ref,
             y_ref, dinv_ref):
    deg = jnp.sum(deg_ref[...], axis=1) + 1.0
    dinv = lax.rsqrt(deg)[:, None]
    x = _selu(jnp.dot(lat_ref[...], wlin_ref[...],
                      preferred_element_type=jnp.float32) + blin_ref[...])
    xin = jnp.concatenate([x, pos_ref[...]], axis=1)
    y_ref[...] = jnp.dot(xin, wf0_ref[...],
                         preferred_element_type=jnp.float32) * dinv
    dinv_ref[...] = dinv


def _stage0(latp, posp, deg2, wlin, blin, wf0):
    return pl.pallas_call(
        _a0_body,
        grid=(_GRID,),
        in_specs=[
            pl.BlockSpec((_R, LAT), lambda i: (i, 0)),
            pl.BlockSpec((_R, 8), lambda i: (i, 0)),
            pl.BlockSpec((_R, NW), lambda i: (i, 0)),
            pl.BlockSpec((LAT, HID), lambda i: (0, 0)),
            pl.BlockSpec((1, HID), lambda i: (0, 0)),
            pl.BlockSpec((HID + 8, HID), lambda i: (0, 0)),
        ],
        out_specs=[
            pl.BlockSpec((_R, HID), lambda i: (i, 0)),
            pl.BlockSpec((_R, 1), lambda i: (i, 0)),
        ],
        out_shape=[
            jax.ShapeDtypeStruct((NP, HID), jnp.float32),
            jax.ShapeDtypeStruct((NP, 1), jnp.float32),
        ],
    )(latp, posp, deg2, wlin, blin, wf0)


def _mid_body(u_ref, y_ref, dinv_ref, b_ref, wfn_ref, pos_ref, yn_ref):
    dinv = dinv_ref[...]
    z = dinv * (u_ref[0] + u_ref[1] + y_ref[...]) + b_ref[...]
    x = _selu(z)
    xin = jnp.concatenate([x, pos_ref[...]], axis=1)
    yn_ref[...] = jnp.dot(xin, wfn_ref[...],
                          preferred_element_type=jnp.float32) * dinv


def _stage_mid(u, y, dinv, b, wfn, posp, DN):
    return pl.pallas_call(
        _mid_body,
        grid=(_GRID,),
        in_specs=[
            pl.BlockSpec((NC, _R, HID), lambda i: (0, i, 0)),
            pl.BlockSpec((_R, HID), lambda i: (i, 0)),
            pl.BlockSpec((_R, 1), lambda i: (i, 0)),
            pl.BlockSpec((1, HID), lambda i: (0, 0)),
            pl.BlockSpec((HID + 8, DN), lambda i: (0, 0)),
            pl.BlockSpec((_R, 8), lambda i: (i, 0)),
        ],
        out_specs=pl.BlockSpec((_R, DN), lambda i: (i, 0)),
        out_shape=jax.ShapeDtypeStruct((NP, DN), jnp.float32),
    )(u, y, dinv, b, wfn, posp)


def _fin_body(u_ref, y_ref, dinv_ref, b_ref, z_ref):
    z_ref[...] = dinv_ref[...] * (u_ref[0] + u_ref[1] + y_ref[...]) \
        + b_ref[...]


def _stage_fin(u, y, dinv, b):
    return pl.pallas_call(
        _fin_body,
        grid=(_GRID,),
        in_specs=[
            pl.BlockSpec((NC, _R, HID), lambda i: (0, i, 0)),
            pl.BlockSpec((_R, HID), lambda i: (i, 0)),
            pl.BlockSpec((_R, 1), lambda i: (i, 0)),
            pl.BlockSpec((1, HID), lambda i: (0, 0)),
        ],
        out_specs=pl.BlockSpec((_R, HID), lambda i: (i, 0)),
        out_shape=jax.ShapeDtypeStruct((NP, HID), jnp.float32),
    )(u, y, dinv, b)


def _pad_w(W, DN):
    return jnp.pad(W, ((0, 8 - DIM), (0, DN - W.shape[1])))


def kernel(latent, edge_index_list, pos_list, W_lin, b_lin, W0, b0, W1, b1,
           W2, b2):
    ei = edge_index_list[0].astype(jnp.int32)
    pos = pos_list[0]
    cap0 = NS * CH0 * K
    cap1 = NS * CH1 * K

    def _region(row, fill, start, stop, cap):
        seg = row[start:stop]
        return jnp.concatenate(
            [seg, jnp.full((cap - (stop - start),), fill, jnp.int32)])

    src = jnp.concatenate([
        _region(ei[0], 0, 0, _S0, cap0),
        _region(ei[0], 0, _S0, E, cap1),
        jnp.zeros((CHM * K,), jnp.int32),
    ])
    dst2 = jnp.concatenate([
        _region(ei[1], N, 0, _S0, cap0),
        _region(ei[1], N, _S0, E, cap1),
        jnp.full((CHM * K,), N, jnp.int32),
    ]).reshape(ECHX, K)
    latp = jnp.pad(latent, ((0, NP - N), (0, 0)))
    posp = jnp.pad(pos, ((0, NP - N), (0, 8 - DIM)))
    wf0 = _pad_w(W0, HID)
    wf1 = _pad_w(W1, HID)
    wf2 = _pad_w(W2, HID)
    b2p = jnp.pad(b2, (0, HID - OUT))

    deg2 = _deg_kernel(dst2).T
    y0, dinv = _stage0(latp, posp, deg2, W_lin, b_lin.reshape(1, HID), wf0)
    u0 = _agg128(y0, src, dst2)
    y1 = _stage_mid(u0, y0, dinv, b0.reshape(1, HID), wf1, posp, HID)
    u1 = _agg128(y1, src, dst2)
    y2 = _stage_mid(u1, y1, dinv, b1.reshape(1, HID), wf2, posp, HID)
    u2 = _agg128(y2, src, dst2)
    z = _stage_fin(u2, y2, dinv, b2p.reshape(1, HID))
    return z[:N, :OUT]


# speedup vs baseline: 1.4551x; 1.4551x over previous
import functools

import jax
import jax.numpy as jnp
from jax import lax
from jax.experimental import pallas as pl
from jax.experimental.pallas import tpu as pltpu
import jax.experimental.pallas.tpu_sc as plsc

N = 10000
E = 320000
DIM = 3
HID = 128
LAT = 128
OUT = 3

NP = 10112
NC, NS = 2, 16
NW = NC * NS
K = 128
RPT = NP // NS

FRAC0 = 0.5


def _ceil4(x):
    return max(4, -(-x // 4) * 4)


_S0 = int(round(FRAC0 * E))
CH0 = _ceil4(-(-_S0 // (NS * K)))
CH1 = _ceil4(-(-(E - _S0) // (NS * K)))
CHM = max(CH0, CH1)
ECH = NS * (CH0 + CH1)
ECHX = ECH + CHM

@functools.lru_cache(maxsize=None)
def _mesh():
    return plsc.VectorSubcoreMesh(core_axis_name="c", subcore_axis_name="s",
                                  num_cores=NC, num_subcores=NS)


def _selu(x):
    alpha = 1.6732632423543772848170429916717
    scale = 1.0507009873554804934193349852946
    return scale * jnp.where(x > 0, x, alpha * (jnp.exp(x) - 1.0))


@functools.lru_cache(maxsize=None)
def _deg_kernel_build():
    return pl.kernel(
        _deg_body,
        out_type=jax.ShapeDtypeStruct((NW, NP), jnp.float32),
        mesh=_mesh(),
        scratch_types=[
            pltpu.VMEM((NP,), jnp.float32),
            pltpu.VMEM((CHM, K), jnp.int32),
        ],
        compiler_params=pltpu.CompilerParams(needs_layout_passes=False),
    )


def _deg_kernel(dst2):
    return _deg_kernel_build()(dst2)


def _chunk_base(c, s):
    return jnp.where(c == 0, s * CH0, NS * CH0 + s * CH1)


def _deg_body(dst_hbm, out_hbm, hist_v, dstv):
    c = lax.axis_index("c")
    s = lax.axis_index("s")
    wid = c * NS + s
    cbase = _chunk_base(c, s)
    chn = jnp.where(c == 0, CH0, CH1)

    pltpu.sync_copy(dst_hbm.at[pl.ds(cbase, CHM)], dstv)

    def zb(k, _):
        hist_v[pl.ds(k * 16, 16)] = jnp.zeros((16,), jnp.float32)
        return 0

    lax.fori_loop(0, NP // 16, zb, 0)
    ones = jnp.ones((16,), jnp.float32)

    kg = K // 16

    def hb(k, _):
        i = lax.div(k, kg)
        j = lax.mul(lax.rem(k, kg), 16)
        idx = dstv[i, pl.ds(j, 16)]
        plsc.addupdate_scatter(hist_v, [idx], ones)
        return 0

    lax.fori_loop(0, chn * kg, hb, 0)
    pltpu.sync_copy(hist_v, out_hbm.at[wid])


@functools.lru_cache(maxsize=None)
def _make_agg(D):
    G = D // 16

    def agg_body(y_hbm, src_hbm, dst_hbm, out_hbm, acc_sh, sidx, didx,
                 rows0, rows1, isem0, isem1, isem2, isem3, gsem0, gsem1):
        c = lax.axis_index("c")
        s = lax.axis_index("s")
        cbase = _chunk_base(c, s)
        chn = jnp.where(c == 0, CH0, CH1)
        isem = (isem0, isem1, isem2, isem3)
        gsem = (gsem0, gsem1)
        rows = (rows0, rows1)

        def issue_idx(ch, b):
            pltpu.async_copy(
                src_hbm.at[pl.ds(lax.mul(cbase + ch, K), K)],
                sidx.at[b], isem[b])
            pltpu.async_copy(dst_hbm.at[cbase + ch], didx.at[b], isem[b])

        def wait_idx(b):
            pltpu.make_async_copy(
                src_hbm.at[pl.ds(0, K)], sidx.at[b], isem[b]).wait()
            pltpu.make_async_copy(
                dst_hbm.at[0], didx.at[b], isem[b]).wait()

        def issue_gather(b4, rb):
            pltpu.async_copy(y_hbm.at[sidx.at[b4]], rows[rb], gsem[rb])

        def wait_gather(b4, rb):
            pltpu.make_async_copy(
                y_hbm.at[sidx.at[b4]], rows[rb], gsem[rb]).wait()

        for b in range(4):
            issue_idx(b, b)

        def zb(k, _):
            i = lax.div(k, G)
            j = lax.mul(lax.rem(k, G), 16)
            rows0[i, pl.ds(j, 16)] = jnp.zeros((16,), jnp.float32)
            return 0

        lax.fori_loop(0, K * G, zb, 0)
        r0 = s * RPT
        for t in range(RPT // K):
            pltpu.sync_copy(rows0, acc_sh.at[pl.ds(r0 + t * K, K)])
        rem = RPT % K
        if rem:
            pltpu.sync_copy(rows0.at[pl.ds(0, rem)],
                            acc_sh.at[pl.ds(r0 + (RPT // K) * K, rem)])
        plsc.subcore_barrier()

        wait_idx(0)
        issue_gather(0, 0)
        wait_idx(1)
        issue_gather(1, 1)

        def body(p, _):
            for b in range(4):
                ch = 4 * p + b
                rb = b & 1
                wait_gather(b, rb)
                pltpu.sync_copy(rows[rb], acc_sh.at[didx.at[b]],
                                add=True)

                @pl.when(ch + 2 < chn)
                def _():
                    b2 = (b + 2) & 3
                    wait_idx(b2)
                    issue_gather(b2, rb)

                @pl.when(ch + 4 < chn)
                def _():
                    issue_idx(ch + 4, b)
            return 0

        lax.fori_loop(0, lax.div(chn, 4), body, 0)
        plsc.subcore_barrier()
        off = 0
        for sz in (K, K, K, K, RPT - 4 * K):
            pltpu.sync_copy(acc_sh.at[pl.ds(r0 + off, sz)],
                            rows0.at[pl.ds(0, sz)])
            pltpu.sync_copy(rows0.at[pl.ds(0, sz)],
                            out_hbm.at[c].at[pl.ds(r0 + off, sz)])
            off += sz

    return pl.kernel(
        agg_body,
        out_type=jax.ShapeDtypeStruct((NC, NP, D), jnp.float32),
        mesh=_mesh(),
        scratch_types=[
            pltpu.VMEM_SHARED((NP, D), jnp.float32),
            pltpu.VMEM((4, K), jnp.int32),
            pltpu.VMEM((4, K), jnp.int32),
            pltpu.VMEM((K, D), jnp.float32),
            pltpu.VMEM((K, D), jnp.float32),
            pltpu.SemaphoreType.DMA,
            pltpu.SemaphoreType.DMA,
            pltpu.SemaphoreType.DMA,
            pltpu.SemaphoreType.DMA,
            pltpu.SemaphoreType.DMA,
            pltpu.SemaphoreType.DMA,
        ],
    )


def _agg128(y, src, dst):
    return _make_agg(128)(y, src, dst)


def _agg16(y, src, dst):
    return _make_agg(16)(y, src, dst)


_R = 1264
_GRID = NP // _R


def _a0_body(lat_ref, pos_ref, deg_ref, wlin_ref, blin_ref, wf0_ref,
             y_ref, dinv_ref):
    deg = jnp.sum(deg_ref[...], axis=1) + 1.0
    dinv = lax.rsqrt(deg)[:, None]
    x = _selu(jnp.dot(lat_ref[...], wlin_ref[...],
                      preferred_element_type=jnp.float32) + blin_ref[...])
    xin = jnp.concatenate([x, pos_ref[...]], axis=1)
    y_ref[...] = jnp.dot(xin, wf0_ref[...],
                         preferred_element_type=jnp.float32) * dinv
    dinv_ref[...] = dinv


def _stage0(latp, posp, deg2, wlin, blin, wf0):
    return pl.pallas_call(
        _a0_body,
        grid=(_GRID,),
        in_specs=[
            pl.BlockSpec((_R, LAT), lambda i: (i, 0)),
            pl.BlockSpec((_R, 8), lambda i: (i, 0)),
            pl.BlockSpec((_R, NW), lambda i: (i, 0)),
            pl.BlockSpec((LAT, HID), lambda i: (0, 0)),
            pl.BlockSpec((1, HID), lambda i: (0, 0)),
            pl.BlockSpec((HID + 8, HID), lambda i: (0, 0)),
        ],
        out_specs=[
            pl.BlockSpec((_R, HID), lambda i: (i, 0)),
            pl.BlockSpec((_R, 1), lambda i: (i, 0)),
        ],
        out_shape=[
            jax.ShapeDtypeStruct((NP, HID), jnp.float32),
            jax.ShapeDtypeStruct((NP, 1), jnp.float32),
        ],
    )(latp, posp, deg2, wlin, blin, wf0)


def _mid_body(u_ref, y_ref, dinv_ref, b_ref, wfn_ref, pos_ref, yn_ref):
    dinv = dinv_ref[...]
    z = dinv * (u_ref[0] + u_ref[1] + y_ref[...]) + b_ref[...]
    x = _selu(z)
    xin = jnp.concatenate([x, pos_ref[...]], axis=1)
    yn_ref[...] = jnp.dot(xin, wfn_ref[...],
                          preferred_element_type=jnp.float32) * dinv


def _stage_mid(u, y, dinv, b, wfn, posp, DN):
    return pl.pallas_call(
        _mid_body,
        grid=(_GRID,),
        in_specs=[
            pl.BlockSpec((NC, _R, HID), lambda i: (0, i, 0)),
            pl.BlockSpec((_R, HID), lambda i: (i, 0)),
            pl.BlockSpec((_R, 1), lambda i: (i, 0)),
            pl.BlockSpec((1, HID), lambda i: (0, 0)),
            pl.BlockSpec((HID + 8, DN), lambda i: (0, 0)),
            pl.BlockSpec((_R, 8), lambda i: (i, 0)),
        ],
        out_specs=pl.BlockSpec((_R, DN), lambda i: (i, 0)),
        out_shape=jax.ShapeDtypeStruct((NP, DN), jnp.float32),
    )(u, y, dinv, b, wfn, posp)


def _fin_body(u_ref, y_ref, dinv_ref, b_ref, z_ref):
    z_ref[...] = dinv_ref[...] * (u_ref[0] + u_ref[1] + y_ref[...]) \
        + b_ref[...]


def _stage_fin(u, y, dinv, b):
    return pl.pallas_call(
        _fin_body,
        grid=(_GRID,),
        in_specs=[
            pl.BlockSpec((NC, _R, HID), lambda i: (0, i, 0)),
            pl.BlockSpec((_R, HID), lambda i: (i, 0)),
            pl.BlockSpec((_R, 1), lambda i: (i, 0)),
            pl.BlockSpec((1, HID), lambda i: (0, 0)),
        ],
        out_specs=pl.BlockSpec((_R, HID), lambda i: (i, 0)),
        out_shape=jax.ShapeDtypeStruct((NP, HID), jnp.float32),
    )(u, y, dinv, b)


def _pad_w(W, DN):
    return jnp.pad(W, ((0, 8 - DIM), (0, DN - W.shape[1])))


def kernel(latent, edge_index_list, pos_list, W_lin, b_lin, W0, b0, W1, b1,
           W2, b2):
    ei = edge_index_list[0].astype(jnp.int32)
    pos = pos_list[0]
    cap0 = NS * CH0 * K
    cap1 = NS * CH1 * K

    def _region(row, fill, start, stop, cap):
        seg = row[start:stop]
        return jnp.concatenate(
            [seg, jnp.full((cap - (stop - start),), fill, jnp.int32)])

    src = jnp.concatenate([
        _region(ei[0], 0, 0, _S0, cap0),
        _region(ei[0] + NP, NP, _S0, E, cap1),
        jnp.zeros((CHM * K,), jnp.int32),
    ])
    dst2 = jnp.concatenate([
        _region(ei[1], N, 0, _S0, cap0),
        _region(ei[1], N, _S0, E, cap1),
        jnp.full((CHM * K,), N, jnp.int32),
    ]).reshape(ECHX, K)
    latp = jnp.pad(latent, ((0, NP - N), (0, 0)))
    posp = jnp.pad(pos, ((0, NP - N), (0, 8 - DIM)))
    wf0 = _pad_w(W0, HID)
    wf1 = _pad_w(W1, HID)
    wf2 = _pad_w(W2, HID)
    b2p = jnp.pad(b2, (0, HID - OUT))

    def dup(y):
        return jnp.concatenate([y, y], axis=0)

    deg2 = _deg_kernel(dst2).T
    y0, dinv = _stage0(latp, posp, deg2, W_lin, b_lin.reshape(1, HID), wf0)
    u0 = _agg128(dup(y0), src, dst2)
    y1 = _stage_mid(u0, y0, dinv, b0.reshape(1, HID), wf1, posp, HID)
    u1 = _agg128(dup(y1), src, dst2)
    y2 = _stage_mid(u1, y1, dinv, b1.reshape(1, HID), wf2, posp, HID)
    u2 = _agg128(dup(y2), src, dst2)
    z = _stage_fin(u2, y2, dinv, b2p.reshape(1, HID))
    return z[:N, :OUT]


# speedup vs baseline: 1.4882x; 1.0228x over previous
import functools

import jax
import jax.numpy as jnp
from jax import lax
from jax.experimental import pallas as pl
from jax.experimental.pallas import tpu as pltpu
import jax.experimental.pallas.tpu_sc as plsc

N = 10000
E = 320000
DIM = 3
HID = 128
LAT = 128
OUT = 3

NP = 10112
NC, NS = 2, 16
NW = NC * NS
K = 128
RPT = NP // NS

FRAC0 = 0.5


def _ceil4(x):
    return max(4, -(-x // 4) * 4)


_S0 = int(round(FRAC0 * E))
CH0 = _ceil4(-(-_S0 // (NS * K)))
CH1 = _ceil4(-(-(E - _S0) // (NS * K)))
CHM = max(CH0, CH1)
ECH = NS * (CH0 + CH1)
ECHX = ECH + CHM

@functools.lru_cache(maxsize=None)
def _mesh():
    return plsc.VectorSubcoreMesh(core_axis_name="c", subcore_axis_name="s",
                                  num_cores=NC, num_subcores=NS)


def _selu(x):
    alpha = 1.6732632423543772848170429916717
    scale = 1.0507009873554804934193349852946
    return scale * jnp.where(x > 0, x, alpha * (jnp.exp(x) - 1.0))


@functools.lru_cache(maxsize=None)
def _deg_kernel_build():
    return pl.kernel(
        _deg_body,
        out_type=jax.ShapeDtypeStruct((NW, NP), jnp.float32),
        mesh=_mesh(),
        scratch_types=[
            pltpu.VMEM((NP,), jnp.float32),
            pltpu.VMEM((CHM, K), jnp.int32),
        ],
        compiler_params=pltpu.CompilerParams(needs_layout_passes=False),
    )


def _deg_kernel(dst2):
    return _deg_kernel_build()(dst2)


def _chunk_base(c, s):
    return jnp.where(c == 0, s * CH0, NS * CH0 + s * CH1)


def _deg_body(dst_hbm, out_hbm, hist_v, dstv):
    c = lax.axis_index("c")
    s = lax.axis_index("s")
    wid = c * NS + s
    cbase = _chunk_base(c, s)
    chn = jnp.where(c == 0, CH0, CH1)

    pltpu.sync_copy(dst_hbm.at[pl.ds(cbase, CHM)], dstv)

    def zb(k, _):
        hist_v[pl.ds(k * 16, 16)] = jnp.zeros((16,), jnp.float32)
        return 0

    lax.fori_loop(0, NP // 16, zb, 0)
    ones = jnp.ones((16,), jnp.float32)

    kg = K // 16

    def hb(k, _):
        i = lax.div(k, kg)
        j = lax.mul(lax.rem(k, kg), 16)
        idx = dstv[i, pl.ds(j, 16)]
        plsc.addupdate_scatter(hist_v, [idx], ones)
        return 0

    lax.fori_loop(0, chn * kg, hb, 0)
    pltpu.sync_copy(hist_v, out_hbm.at[wid])


@functools.lru_cache(maxsize=None)
def _make_agg(D):
    G = D // 16

    def agg_body(y_hbm, src_hbm, dst_hbm, out_hbm, acc_sh, sidx, didx,
                 rows0, rows1, isem0, isem1, isem2, isem3, gsem0, gsem1):
        c = lax.axis_index("c")
        s = lax.axis_index("s")
        cbase = _chunk_base(c, s)
        chn = jnp.where(c == 0, CH0, CH1)
        isem = (isem0, isem1, isem2, isem3)
        gsem = (gsem0, gsem1)
        rows = (rows0, rows1)

        def issue_idx(ch, b):
            pltpu.async_copy(
                src_hbm.at[pl.ds(lax.mul(cbase + ch, K), K)],
                sidx.at[b], isem[b])
            pltpu.async_copy(dst_hbm.at[cbase + ch], didx.at[b], isem[b])

        def wait_idx(b):
            pltpu.make_async_copy(
                src_hbm.at[pl.ds(0, K)], sidx.at[b], isem[b]).wait()
            pltpu.make_async_copy(
                dst_hbm.at[0], didx.at[b], isem[b]).wait()

        def issue_gather(b4, rb):
            pltpu.async_copy(y_hbm.at[sidx.at[b4]], rows[rb], gsem[rb])

        def wait_gather(b4, rb):
            pltpu.make_async_copy(
                y_hbm.at[sidx.at[b4]], rows[rb], gsem[rb]).wait()

        for b in range(4):
            issue_idx(b, b)

        def zb(k, _):
            i = lax.div(k, G)
            j = lax.mul(lax.rem(k, G), 16)
            rows0[i, pl.ds(j, 16)] = jnp.zeros((16,), jnp.float32)
            return 0

        lax.fori_loop(0, K * G, zb, 0)
        r0 = s * RPT
        for t in range(RPT // K):
            pltpu.sync_copy(rows0, acc_sh.at[pl.ds(r0 + t * K, K)])
        rem = RPT % K
        if rem:
            pltpu.sync_copy(rows0.at[pl.ds(0, rem)],
                            acc_sh.at[pl.ds(r0 + (RPT // K) * K, rem)])
        plsc.subcore_barrier()

        wait_idx(0)
        issue_gather(0, 0)
        wait_idx(1)
        issue_gather(1, 1)

        def body(p, _):
            for b in range(4):
                ch = 4 * p + b
                rb = b & 1
                wait_gather(b, rb)
                pltpu.sync_copy(rows[rb], acc_sh.at[didx.at[b]],
                                add=True)

                @pl.when(ch + 2 < chn)
                def _():
                    b2 = (b + 2) & 3
                    wait_idx(b2)
                    issue_gather(b2, rb)

                @pl.when(ch + 4 < chn)
                def _():
                    issue_idx(ch + 4, b)
            return 0

        lax.fori_loop(0, lax.div(chn, 4), body, 0)
        plsc.subcore_barrier()
        off = 0
        for sz in (K, K, K, K, RPT - 4 * K):
            pltpu.sync_copy(acc_sh.at[pl.ds(r0 + off, sz)],
                            rows0.at[pl.ds(0, sz)])
            pltpu.sync_copy(rows0.at[pl.ds(0, sz)],
                            out_hbm.at[c].at[pl.ds(r0 + off, sz)])
            off += sz

    return pl.kernel(
        agg_body,
        out_type=jax.ShapeDtypeStruct((NC, NP, D), jnp.float32),
        mesh=_mesh(),
        scratch_types=[
            pltpu.VMEM_SHARED((NP, D), jnp.float32),
            pltpu.VMEM((4, K), jnp.int32),
            pltpu.VMEM((4, K), jnp.int32),
            pltpu.VMEM((K, D), jnp.float32),
            pltpu.VMEM((K, D), jnp.float32),
            pltpu.SemaphoreType.DMA,
            pltpu.SemaphoreType.DMA,
            pltpu.SemaphoreType.DMA,
            pltpu.SemaphoreType.DMA,
            pltpu.SemaphoreType.DMA,
            pltpu.SemaphoreType.DMA,
        ],
    )


def _agg128(y, src, dst):
    return _make_agg(128)(y, src, dst)


def _agg16(y, src, dst):
    return _make_agg(16)(y, src, dst)


_R = 1264
_GRID = NP // _R


def _a0_body(lat_ref, pos_ref, deg_ref, wlin_ref, blin_ref, wf0_ref,
             y_ref, dinv_ref):
    deg = jnp.sum(deg_ref[...], axis=1) + 1.0
    dinv = lax.rsqrt(deg)[:, None]
    x = _selu(jnp.dot(lat_ref[...], wlin_ref[...],
                      preferred_element_type=jnp.float32) + blin_ref[...])
    xin = jnp.concatenate([x, pos_ref[...]], axis=1)
    y_ref[...] = jnp.dot(xin, wf0_ref[...],
                         preferred_element_type=jnp.float32) * dinv
    dinv_ref[...] = dinv


def _stage0(latp, posp, deg2, wlin, blin, wf0):
    return pl.pallas_call(
        _a0_body,
        grid=(_GRID,),
        in_specs=[
            pl.BlockSpec((_R, LAT), lambda i: (i, 0)),
            pl.BlockSpec((_R, 8), lambda i: (i, 0)),
            pl.BlockSpec((_R, NW), lambda i: (i, 0)),
            pl.BlockSpec((LAT, HID), lambda i: (0, 0)),
            pl.BlockSpec((1, HID), lambda i: (0, 0)),
            pl.BlockSpec((HID + 8, HID), lambda i: (0, 0)),
        ],
        out_specs=[
            pl.BlockSpec((_R, HID), lambda i: (i, 0)),
            pl.BlockSpec((_R, 1), lambda i: (i, 0)),
        ],
        out_shape=[
            jax.ShapeDtypeStruct((NP, HID), jnp.float32),
            jax.ShapeDtypeStruct((NP, 1), jnp.float32),
        ],
    )(latp, posp, deg2, wlin, blin, wf0)


def _mid_body(u_ref, y_ref, dinv_ref, b_ref, wfn_ref, pos_ref, yn_ref):
    dinv = dinv_ref[...]
    z = dinv * (u_ref[0] + u_ref[1] + y_ref[...]) + b_ref[...]
    x = _selu(z)
    xin = jnp.concatenate([x, pos_ref[...]], axis=1)
    yn_ref[...] = jnp.dot(xin, wfn_ref[...],
                          preferred_element_type=jnp.float32) * dinv


def _stage_mid(u, y, dinv, b, wfn, posp, DN):
    return pl.pallas_call(
        _mid_body,
        grid=(_GRID,),
        in_specs=[
            pl.BlockSpec((NC, _R, HID), lambda i: (0, i, 0)),
            pl.BlockSpec((_R, HID), lambda i: (i, 0)),
            pl.BlockSpec((_R, 1), lambda i: (i, 0)),
            pl.BlockSpec((1, HID), lambda i: (0, 0)),
            pl.BlockSpec((HID + 8, DN), lambda i: (0, 0)),
            pl.BlockSpec((_R, 8), lambda i: (i, 0)),
        ],
        out_specs=pl.BlockSpec((_R, DN), lambda i: (i, 0)),
        out_shape=jax.ShapeDtypeStruct((NP, DN), jnp.float32),
    )(u, y, dinv, b, wfn, posp)


def _fin_body(u_ref, y_ref, dinv_ref, b_ref, z_ref):
    z_ref[...] = dinv_ref[...] * (u_ref[0] + u_ref[1] + y_ref[...]) \
        + b_ref[...]


def _stage_fin(u, y, dinv, b):
    return pl.pallas_call(
        _fin_body,
        grid=(_GRID,),
        in_specs=[
            pl.BlockSpec((NC, _R, HID), lambda i: (0, i, 0)),
            pl.BlockSpec((_R, HID), lambda i: (i, 0)),
            pl.BlockSpec((_R, 1), lambda i: (i, 0)),
            pl.BlockSpec((1, HID), lambda i: (0, 0)),
        ],
        out_specs=pl.BlockSpec((_R, HID), lambda i: (i, 0)),
        out_shape=jax.ShapeDtypeStruct((NP, HID), jnp.float32),
    )(u, y, dinv, b)


def _pad_w(W, DN):
    return jnp.pad(W, ((0, 8 - DIM), (0, DN - W.shape[1])))


def kernel(latent, edge_index_list, pos_list, W_lin, b_lin, W0, b0, W1, b1,
           W2, b2):
    ei = edge_index_list[0].astype(jnp.int32)
    pos = pos_list[0]
    cap0 = NS * CH0 * K
    cap1 = NS * CH1 * K

    def _region(row, fill, start, stop, cap):
        seg = row[start:stop]
        return jnp.concatenate(
            [seg, jnp.full((cap - (stop - start),), fill, jnp.int32)])

    alt0 = (jnp.arange(_S0, dtype=jnp.int32) % 2) * NP
    alt1 = (jnp.arange(E - _S0, dtype=jnp.int32) % 2) * NP
    src = jnp.concatenate([
        _region(ei[0].at[:_S0].add(alt0), 0, 0, _S0, cap0),
        _region(ei[0].at[_S0:].add(2 * NP + alt1), NP, _S0, E, cap1),
        jnp.zeros((CHM * K,), jnp.int32),
    ])
    dst2 = jnp.concatenate([
        _region(ei[1], N, 0, _S0, cap0),
        _region(ei[1], N, _S0, E, cap1),
        jnp.full((CHM * K,), N, jnp.int32),
    ]).reshape(ECHX, K)
    latp = jnp.pad(latent, ((0, NP - N), (0, 0)))
    posp = jnp.pad(pos, ((0, NP - N), (0, 8 - DIM)))
    wf0 = _pad_w(W0, HID)
    wf1 = _pad_w(W1, HID)
    wf2 = _pad_w(W2, HID)
    b2p = jnp.pad(b2, (0, HID - OUT))

    def dup(y):
        return jnp.concatenate([y, y, y, y], axis=0)

    deg2 = _deg_kernel(dst2).T
    y0, dinv = _stage0(latp, posp, deg2, W_lin, b_lin.reshape(1, HID), wf0)
    u0 = _agg128(dup(y0), src, dst2)
    y1 = _stage_mid(u0, y0, dinv, b0.reshape(1, HID), wf1, posp, HID)
    u1 = _agg128(dup(y1), src, dst2)
    y2 = _stage_mid(u1, y1, dinv, b1.reshape(1, HID), wf2, posp, HID)
    u2 = _agg128(dup(y2), src, dst2)
    z = _stage_fin(u2, y2, dinv, b2p.reshape(1, HID))
    return z[:N, :OUT]


# speedup vs baseline: 1.4915x; 1.0022x over previous
import functools

import jax
import jax.numpy as jnp
from jax import lax
from jax.experimental import pallas as pl
from jax.experimental.pallas import tpu as pltpu
import jax.experimental.pallas.tpu_sc as plsc

N = 10000
E = 320000
DIM = 3
HID = 128
LAT = 128
OUT = 3

NP = 10112
NC, NS = 2, 16
NW = NC * NS
K = 128
RPT = NP // NS

FRAC0 = 0.5


def _ceil4(x):
    return max(4, -(-x // 4) * 4)


_S0 = int(round(FRAC0 * E))
CH0 = _ceil4(-(-_S0 // (NS * K)))
CH1 = _ceil4(-(-(E - _S0) // (NS * K)))
CHM = max(CH0, CH1)
ECH = NS * (CH0 + CH1)
ECHX = ECH + CHM

@functools.lru_cache(maxsize=None)
def _mesh():
    return plsc.VectorSubcoreMesh(core_axis_name="c", subcore_axis_name="s",
                                  num_cores=NC, num_subcores=NS)


def _selu(x):
    alpha = 1.6732632423543772848170429916717
    scale = 1.0507009873554804934193349852946
    return scale * jnp.where(x > 0, x, alpha * (jnp.exp(x) - 1.0))


@functools.lru_cache(maxsize=None)
def _deg_kernel_build():
    return pl.kernel(
        _deg_body,
        out_type=jax.ShapeDtypeStruct((NW, NP), jnp.float32),
        mesh=_mesh(),
        scratch_types=[
            pltpu.VMEM((NP,), jnp.float32),
            pltpu.VMEM((CHM, K), jnp.int32),
        ],
        compiler_params=pltpu.CompilerParams(needs_layout_passes=False),
    )


def _deg_kernel(dst2):
    return _deg_kernel_build()(dst2)


def _chunk_base(c, s):
    return jnp.where(c == 0, s * CH0, NS * CH0 + s * CH1)


def _deg_body(dst_hbm, out_hbm, hist_v, dstv):
    c = lax.axis_index("c")
    s = lax.axis_index("s")
    wid = c * NS + s
    cbase = _chunk_base(c, s)
    chn = jnp.where(c == 0, CH0, CH1)

    pltpu.sync_copy(dst_hbm.at[pl.ds(cbase, CHM)], dstv)

    def zb(k, _):
        hist_v[pl.ds(k * 16, 16)] = jnp.zeros((16,), jnp.float32)
        return 0

    lax.fori_loop(0, NP // 16, zb, 0)
    ones = jnp.ones((16,), jnp.float32)

    kg = K // 16

    def hb(k, _):
        i = lax.div(k, kg)
        j = lax.mul(lax.rem(k, kg), 16)
        idx = dstv[i, pl.ds(j, 16)]
        plsc.addupdate_scatter(hist_v, [idx], ones)
        return 0

    lax.fori_loop(0, chn * kg, hb, 0)
    pltpu.sync_copy(hist_v, out_hbm.at[wid])


@functools.lru_cache(maxsize=None)
def _make_agg(D):
    G = D // 16

    def agg_body(y_hbm, src_hbm, dst_hbm, out_hbm, acc_sh, sidx, didx,
                 rows0, rows1, isem0, isem1, isem2, isem3, gsem0, gsem1):
        c = lax.axis_index("c")
        s = lax.axis_index("s")
        cbase = _chunk_base(c, s)
        chn = jnp.where(c == 0, CH0, CH1)
        isem = (isem0, isem1, isem2, isem3)
        gsem = (gsem0, gsem1)
        rows = (rows0, rows1)

        def issue_idx(ch, b):
            pltpu.async_copy(
                src_hbm.at[pl.ds(lax.mul(cbase + ch, K), K)],
                sidx.at[b], isem[b])
            pltpu.async_copy(dst_hbm.at[cbase + ch], didx.at[b], isem[b])

        def wait_idx(b):
            pltpu.make_async_copy(
                src_hbm.at[pl.ds(0, K)], sidx.at[b], isem[b]).wait()
            pltpu.make_async_copy(
                dst_hbm.at[0], didx.at[b], isem[b]).wait()

        def issue_gather(b4, rb):
            pltpu.async_copy(y_hbm.at[sidx.at[b4]], rows[rb], gsem[rb])

        def wait_gather(b4, rb):
            pltpu.make_async_copy(
                y_hbm.at[sidx.at[b4]], rows[rb], gsem[rb]).wait()

        for b in range(4):
            issue_idx(b, b)

        def zb(k, _):
            i = lax.div(k, G)
            j = lax.mul(lax.rem(k, G), 16)
            rows0[i, pl.ds(j, 16)] = jnp.zeros((16,), jnp.float32)
            return 0

        lax.fori_loop(0, K * G, zb, 0)
        r0 = s * RPT
        for t in range(RPT // K):
            pltpu.sync_copy(rows0, acc_sh.at[pl.ds(r0 + t * K, K)])
        rem = RPT % K
        if rem:
            pltpu.sync_copy(rows0.at[pl.ds(0, rem)],
                            acc_sh.at[pl.ds(r0 + (RPT // K) * K, rem)])
        plsc.subcore_barrier()

        wait_idx(0)
        issue_gather(0, 0)
        wait_idx(1)
        issue_gather(1, 1)

        def body(p, _):
            for b in range(4):
                ch = 4 * p + b
                rb = b & 1
                wait_gather(b, rb)
                pltpu.sync_copy(rows[rb], acc_sh.at[didx.at[b]],
                                add=True)

                @pl.when(ch + 2 < chn)
                def _():
                    b2 = (b + 2) & 3
                    wait_idx(b2)
                    issue_gather(b2, rb)

                @pl.when(ch + 4 < chn)
                def _():
                    issue_idx(ch + 4, b)
            return 0

        lax.fori_loop(0, lax.div(chn, 4), body, 0)
        plsc.subcore_barrier()
        off = 0
        for sz in (K, K, K, K, RPT - 4 * K):
            pltpu.sync_copy(acc_sh.at[pl.ds(r0 + off, sz)],
                            rows0.at[pl.ds(0, sz)])
            pltpu.sync_copy(rows0.at[pl.ds(0, sz)],
                            out_hbm.at[c].at[pl.ds(r0 + off, sz)])
            off += sz

    return pl.kernel(
        agg_body,
        out_type=jax.ShapeDtypeStruct((NC, NP, D), jnp.float32),
        mesh=_mesh(),
        scratch_types=[
            pltpu.VMEM_SHARED((NP, D), jnp.float32),
            pltpu.VMEM((4, K), jnp.int32),
            pltpu.VMEM((4, K), jnp.int32),
            pltpu.VMEM((K, D), jnp.float32),
            pltpu.VMEM((K, D), jnp.float32),
            pltpu.SemaphoreType.DMA,
            pltpu.SemaphoreType.DMA,
            pltpu.SemaphoreType.DMA,
            pltpu.SemaphoreType.DMA,
            pltpu.SemaphoreType.DMA,
            pltpu.SemaphoreType.DMA,
        ],
    )


def _agg128(y, src, dst):
    return _make_agg(128)(y, src, dst)


def _agg16(y, src, dst):
    return _make_agg(16)(y, src, dst)


_R = 1264
_GRID = NP // _R


def _a0_body(lat_ref, pos_ref, deg_ref, wlin_ref, blin_ref, wf0_ref,
             y_ref, dinv_ref):
    deg = jnp.sum(deg_ref[...], axis=1) + 1.0
    dinv = lax.rsqrt(deg)[:, None]
    x = _selu(jnp.dot(lat_ref[...], wlin_ref[...],
                      preferred_element_type=jnp.float32) + blin_ref[...])
    xin = jnp.concatenate([x, pos_ref[...]], axis=1)
    y_ref[...] = jnp.dot(xin, wf0_ref[...],
                         preferred_element_type=jnp.float32) * dinv
    dinv_ref[...] = dinv


def _stage0(latp, posp, deg2, wlin, blin, wf0):
    return pl.pallas_call(
        _a0_body,
        grid=(_GRID,),
        in_specs=[
            pl.BlockSpec((_R, LAT), lambda i: (i, 0)),
            pl.BlockSpec((_R, 8), lambda i: (i, 0)),
            pl.BlockSpec((_R, NW), lambda i: (i, 0)),
            pl.BlockSpec((LAT, HID), lambda i: (0, 0)),
            pl.BlockSpec((1, HID), lambda i: (0, 0)),
            pl.BlockSpec((HID + 8, HID), lambda i: (0, 0)),
        ],
        out_specs=[
            pl.BlockSpec((_R, HID), lambda i: (i, 0)),
            pl.BlockSpec((_R, 1), lambda i: (i, 0)),
        ],
        out_shape=[
            jax.ShapeDtypeStruct((NP, HID), jnp.float32),
            jax.ShapeDtypeStruct((NP, 1), jnp.float32),
        ],
    )(latp, posp, deg2, wlin, blin, wf0)


def _mid_body(u_ref, y_ref, dinv_ref, b_ref, wfn_ref, pos_ref, yn_ref):
    dinv = dinv_ref[...]
    z = dinv * (u_ref[0] + u_ref[1] + y_ref[...]) + b_ref[...]
    x = _selu(z)
    xin = jnp.concatenate([x, pos_ref[...]], axis=1)
    yn_ref[...] = jnp.dot(xin, wfn_ref[...],
                          preferred_element_type=jnp.float32) * dinv


def _stage_mid(u, y, dinv, b, wfn, posp, DN):
    return pl.pallas_call(
        _mid_body,
        grid=(_GRID,),
        in_specs=[
            pl.BlockSpec((NC, _R, HID), lambda i: (0, i, 0)),
            pl.BlockSpec((_R, HID), lambda i: (i, 0)),
            pl.BlockSpec((_R, 1), lambda i: (i, 0)),
            pl.BlockSpec((1, HID), lambda i: (0, 0)),
            pl.BlockSpec((HID + 8, DN), lambda i: (0, 0)),
            pl.BlockSpec((_R, 8), lambda i: (i, 0)),
        ],
        out_specs=pl.BlockSpec((_R, DN), lambda i: (i, 0)),
        out_shape=jax.ShapeDtypeStruct((NP, DN), jnp.float32),
    )(u, y, dinv, b, wfn, posp)


def _fin_body(u_ref, y_ref, dinv_ref, b_ref, z_ref):
    z_ref[...] = dinv_ref[...] * (u_ref[0] + u_ref[1] + y_ref[...]) \
        + b_ref[...]


def _stage_fin(u, y, dinv, b):
    return pl.pallas_call(
        _fin_body,
        grid=(_GRID,),
        in_specs=[
            pl.BlockSpec((NC, _R, HID), lambda i: (0, i, 0)),
            pl.BlockSpec((_R, HID), lambda i: (i, 0)),
            pl.BlockSpec((_R, 1), lambda i: (i, 0)),
            pl.BlockSpec((1, HID), lambda i: (0, 0)),
        ],
        out_specs=pl.BlockSpec((_R, HID), lambda i: (i, 0)),
        out_shape=jax.ShapeDtypeStruct((NP, HID), jnp.float32),
    )(u, y, dinv, b)


def _pad_w(W, DN):
    return jnp.pad(W, ((0, 8 - DIM), (0, DN - W.shape[1])))


def kernel(latent, edge_index_list, pos_list, W_lin, b_lin, W0, b0, W1, b1,
           W2, b2):
    ei = edge_index_list[0].astype(jnp.int32)
    pos = pos_list[0]
    cap0 = NS * CH0 * K
    cap1 = NS * CH1 * K

    def _region(row, fill, start, stop, cap):
        seg = row[start:stop]
        return jnp.concatenate(
            [seg, jnp.full((cap - (stop - start),), fill, jnp.int32)])

    alt0 = (jnp.arange(_S0, dtype=jnp.int32) % 4) * NP
    alt1 = (jnp.arange(E - _S0, dtype=jnp.int32) % 4) * NP
    src = jnp.concatenate([
        _region(ei[0].at[:_S0].add(alt0), 0, 0, _S0, cap0),
        _region(ei[0].at[_S0:].add(4 * NP + alt1), NP, _S0, E, cap1),
        jnp.zeros((CHM * K,), jnp.int32),
    ])
    dst2 = jnp.concatenate([
        _region(ei[1], N, 0, _S0, cap0),
        _region(ei[1], N, _S0, E, cap1),
        jnp.full((CHM * K,), N, jnp.int32),
    ]).reshape(ECHX, K)
    latp = jnp.pad(latent, ((0, NP - N), (0, 0)))
    posp = jnp.pad(pos, ((0, NP - N), (0, 8 - DIM)))
    wf0 = _pad_w(W0, HID)
    wf1 = _pad_w(W1, HID)
    wf2 = _pad_w(W2, HID)
    b2p = jnp.pad(b2, (0, HID - OUT))

    def dup(y):
        return jnp.concatenate([y] * 8, axis=0)

    deg2 = _deg_kernel(dst2).T
    y0, dinv = _stage0(latp, posp, deg2, W_lin, b_lin.reshape(1, HID), wf0)
    u0 = _agg128(dup(y0), src, dst2)
    y1 = _stage_mid(u0, y0, dinv, b0.reshape(1, HID), wf1, posp, HID)
    u1 = _agg128(dup(y1), src, dst2)
    y2 = _stage_mid(u1, y1, dinv, b1.reshape(1, HID), wf2, posp, HID)
    u2 = _agg128(dup(y2), src, dst2)
    z = _stage_fin(u2, y2, dinv, b2p.reshape(1, HID))
    return z[:N, :OUT]
